# baseline (speedup 1.0000x reference)
import functools

import jax
import jax.numpy as jnp
from jax import lax
from jax.experimental import pallas as pl
from jax.experimental.pallas import tpu as pltpu
from jax.experimental.pallas import tpu_sc as plsc

B = 16384
N_ROWS = 1000000
MF_DIM = 16
MLP_HALF = 32
NC = 2
NS = 16
NW = NC * NS
BPW = B // NW
CH = 128
NCH = BPW // CH


TCOL = 2048
_TGRID = (N_ROWS + TCOL - 1) // TCOL


def _tr_body(mu_ref, mi_ref, fu_ref, fi_ref, muo_ref, mio_ref, fuo_ref, fio_ref):
    muo_ref[...] = jnp.transpose(mu_ref[...])
    mio_ref[...] = jnp.transpose(mi_ref[...])
    fuo_ref[...] = jnp.transpose(fu_ref[...])
    fio_ref[...] = jnp.transpose(fi_ref[...])


def _relayout(mlp_user_t, mlp_item_t, mf_user_t, mf_item_t):
    mlp_spec = pl.BlockSpec((MLP_HALF, TCOL), lambda g: (0, g))
    mf_spec = pl.BlockSpec((MF_DIM, TCOL), lambda g: (0, g))
    mlp_out = pl.BlockSpec((TCOL, MLP_HALF), lambda g: (g, 0))
    mf_out = pl.BlockSpec((TCOL, MF_DIM), lambda g: (g, 0))
    return pl.pallas_call(
        _tr_body,
        grid=(_TGRID,),
        in_specs=[mlp_spec, mlp_spec, mf_spec, mf_spec],
        out_specs=[mlp_out, mlp_out, mf_out, mf_out],
        out_shape=[
            jax.ShapeDtypeStruct((N_ROWS, MLP_HALF), jnp.float32),
            jax.ShapeDtypeStruct((N_ROWS, MLP_HALF), jnp.float32),
            jax.ShapeDtypeStruct((N_ROWS, MF_DIM), jnp.float32),
            jax.ShapeDtypeStruct((N_ROWS, MF_DIM), jnp.float32),
        ],
    )(mlp_user_t, mlp_item_t, mf_user_t, mf_item_t)


_mesh = plsc.VectorSubcoreMesh(core_axis_name="c", subcore_axis_name="s")


@functools.partial(
    pl.kernel,
    mesh=_mesh,
    compiler_params=pltpu.CompilerParams(use_tc_tiling_on_sc=False),
    out_type=[
        jax.ShapeDtypeStruct((B, MLP_HALF), jnp.float32),
        jax.ShapeDtypeStruct((B, MLP_HALF), jnp.float32),
        jax.ShapeDtypeStruct((B, MF_DIM), jnp.float32),
        jax.ShapeDtypeStruct((B, MF_DIM), jnp.float32),
    ],
    scratch_types=[
        pltpu.VMEM((BPW,), jnp.int32),
        pltpu.VMEM((BPW,), jnp.int32),
        pltpu.VMEM((BPW, MLP_HALF), jnp.float32),
        pltpu.VMEM((BPW, MLP_HALF), jnp.float32),
        pltpu.VMEM((BPW, MF_DIM), jnp.float32),
        pltpu.VMEM((BPW, MF_DIM), jnp.float32),
        pltpu.SemaphoreType.DMA,
    ],
)
def _sc_gather(user_ids, item_ids, mlp_user, mlp_item, mf_user, mf_item,
               u_out, i_out, fu_out, fi_out,
               uidx, iidx, urows, irows, furows, firows, sem):
    wid = lax.axis_index("s") * NC + lax.axis_index("c")
    base = wid * BPW
    pltpu.sync_copy(user_ids.at[pl.ds(base, BPW)], uidx)
    pltpu.sync_copy(item_ids.at[pl.ds(base, BPW)], iidx)
    copies = []
    for j in range(NCH):
        sl = pl.ds(j * CH, CH)
        copies.append(pltpu.async_copy(mlp_user.at[uidx.at[sl]], urows.at[sl], sem))
        copies.append(pltpu.async_copy(mlp_item.at[iidx.at[sl]], irows.at[sl], sem))
        copies.append(pltpu.async_copy(mf_user.at[uidx.at[sl]], furows.at[sl], sem))
        copies.append(pltpu.async_copy(mf_item.at[iidx.at[sl]], firows.at[sl], sem))
    for c in copies:
        c.wait()
    pltpu.sync_copy(urows, u_out.at[pl.ds(base, BPW)])
    pltpu.sync_copy(irows, i_out.at[pl.ds(base, BPW)])
    pltpu.sync_copy(furows, fu_out.at[pl.ds(base, BPW)])
    pltpu.sync_copy(firows, fi_out.at[pl.ds(base, BPW)])


BT = 2048


def _tc_body(u_ref, i_ref, fu_ref, fi_ref, W1_ref, b1_ref, W2_ref, b2_ref,
             W3_ref, b3_ref, W4_ref, b4_ref, out_ref):
    x = jnp.concatenate([u_ref[...], i_ref[...]], axis=1)
    h = jnp.maximum(jnp.dot(x, W1_ref[...], preferred_element_type=jnp.float32)
                    + b1_ref[...], 0.0)
    h = jnp.maximum(jnp.dot(h, W2_ref[...], preferred_element_type=jnp.float32)
                    + b2_ref[...], 0.0)
    h = jnp.maximum(jnp.dot(h, W3_ref[...], preferred_element_type=jnp.float32)
                    + b3_ref[...], 0.0)
    mf = jnp.sum(fu_ref[...] * fi_ref[...], axis=1, keepdims=True)
    z = (mf * W4_ref[0:1, :]
         + jnp.dot(h, W4_ref[1:9, :], preferred_element_type=jnp.float32)
         + b4_ref[...])
    out_ref[...] = 1.0 / (1.0 + jnp.exp(-z))


def _tc_mlp(urows, irows, furows, firows, W1, b1r, W2, b2r, W3, b3r, W4p, b4r):
    grid = (B // BT,)
    full = lambda g: (0, 0)
    tile = lambda g: (g, 0)
    return pl.pallas_call(
        _tc_body,
        grid=grid,
        in_specs=[
            pl.BlockSpec((BT, MLP_HALF), tile),
            pl.BlockSpec((BT, MLP_HALF), tile),
            pl.BlockSpec((BT, MF_DIM), tile),
            pl.BlockSpec((BT, MF_DIM), tile),
            pl.BlockSpec((64, 32), full),
            pl.BlockSpec((1, 32), full),
            pl.BlockSpec((32, 16), full),
            pl.BlockSpec((1, 16), full),
            pl.BlockSpec((16, 8), full),
            pl.BlockSpec((1, 8), full),
            pl.BlockSpec((16, 1), full),
            pl.BlockSpec((1, 1), full),
        ],
        out_specs=pl.BlockSpec((BT, 1), tile),
        out_shape=jax.ShapeDtypeStruct((B, 1), jnp.float32),
    )(urows, irows, furows, firows, W1, b1r, W2, b2r, W3, b3r, W4p, b4r)


def kernel(user_ids, item_ids, mf_user, mf_item, mlp_user, mlp_item,
           W1, b1, W2, b2, W3, b3, W4, b4):
    mlp_user_rm, mlp_item_rm, mf_user_rm, mf_item_rm = _relayout(
        mlp_user.T, mlp_item.T, mf_user.T, mf_item.T)
    urows, irows, furows, firows = _sc_gather(
        user_ids, item_ids, mlp_user_rm, mlp_item_rm, mf_user_rm, mf_item_rm)
    W4p = jnp.pad(W4, ((0, 7), (0, 0)))
    return _tc_mlp(urows, irows, furows, firows,
                   W1, b1.reshape(1, 32), W2, b2.reshape(1, 16),
                   W3, b3.reshape(1, 8), W4p, b4.reshape(1, 1))


# speedup vs baseline: 5.5781x; 5.5781x over previous
import functools

import jax
import jax.numpy as jnp
from jax import lax
from jax.experimental import pallas as pl
from jax.experimental.pallas import tpu as pltpu
from jax.experimental.pallas import tpu_sc as plsc

B = 16384
N_ROWS = 1000000
MF_DIM = 16
MLP_HALF = 32
QMLP = 2048 * 122
QMF = 2048 * 61
RMLP = 2048 * 123
RMF = 2048 * 62
NC = 2
NS = 16
NW = NC * NS
BPW = B // NW
CH = 64
NCH = BPW // CH
L = 16


TCOL = 2048


def _tr_mlp_body(mu0, mu1, mu2, mu3, mi0, mi1, mi2, mi3, muo, mio):
    muo[...] = jnp.transpose(jnp.concatenate(
        [r[...] for r in (mu0, mu1, mu2, mu3)], axis=0))
    mio[...] = jnp.transpose(jnp.concatenate(
        [r[...] for r in (mi0, mi1, mi2, mi3)], axis=0))


def _tr_mf_body(fu0, fu1, fu2, fu3, fu4, fu5, fu6, fu7,
                fi0, fi1, fi2, fi3, fi4, fi5, fi6, fi7, fuo, fio):
    fuo[...] = jnp.transpose(jnp.concatenate(
        [r[...] for r in (fu0, fu1, fu2, fu3, fu4, fu5, fu6, fu7)], axis=0))
    fio[...] = jnp.transpose(jnp.concatenate(
        [r[...] for r in (fi0, fi1, fi2, fi3, fi4, fi5, fi6, fi7)], axis=0))


def _relayout(mlp_user_t, mlp_item_t, mf_user_t, mf_item_t):
    ngrid_mlp = RMLP // TCOL
    ngrid_mf = RMF // TCOL
    mlp_specs = [pl.BlockSpec((MLP_HALF, TCOL),
                              lambda g, j=j: (0, g + 122 * j))
                 for j in range(4)]
    mf_specs = [pl.BlockSpec((MF_DIM, TCOL),
                             lambda g, j=j: (0, g + 61 * j))
                for j in range(8)]
    out_row = lambda g: (g, 0)
    tr_params = pltpu.CompilerParams(fuse_transposed_lhs_in_matmul=True)
    mlp_user_p, mlp_item_p = pl.pallas_call(
        _tr_mlp_body,
        grid=(ngrid_mlp,),
        compiler_params=tr_params,
        in_specs=mlp_specs + mlp_specs,
        out_specs=[pl.BlockSpec((TCOL, 128), out_row)] * 2,
        out_shape=[jax.ShapeDtypeStruct((RMLP, 128), jnp.float32)] * 2,
    )(*([mlp_user_t] * 4 + [mlp_item_t] * 4))
    mf_user_p, mf_item_p = pl.pallas_call(
        _tr_mf_body,
        grid=(ngrid_mf,),
        compiler_params=tr_params,
        in_specs=mf_specs + mf_specs,
        out_specs=[pl.BlockSpec((TCOL, 128), out_row)] * 2,
        out_shape=[jax.ShapeDtypeStruct((RMF, 128), jnp.float32)] * 2,
    )(*([mf_user_t] * 8 + [mf_item_t] * 8))
    return mlp_user_p, mlp_item_p, mf_user_p, mf_item_p


_mesh = plsc.VectorSubcoreMesh(core_axis_name="c", subcore_axis_name="s")


def _iota16():
    return lax.iota(jnp.int32, L)


def _split_mlp(ids):
    j = ((ids >= QMLP).astype(jnp.int32) + (ids >= 2 * QMLP).astype(jnp.int32)
         + (ids >= 3 * QMLP).astype(jnp.int32))
    return ids - j * QMLP, j * MLP_HALF


def _split_mf(ids):
    j = (ids >= QMF).astype(jnp.int32)
    for k in range(2, 8):
        j = j + (ids >= k * QMF).astype(jnp.int32)
    return ids - j * QMF, j * MF_DIM


@functools.partial(
    pl.kernel,
    mesh=_mesh,
    compiler_params=pltpu.CompilerParams(needs_layout_passes=False),
    out_type=jax.ShapeDtypeStruct((B, 128), jnp.float32),
    scratch_types=[
        pltpu.VMEM((BPW,), jnp.int32),
        pltpu.VMEM((BPW,), jnp.int32),
        pltpu.VMEM((BPW,), jnp.int32),
        pltpu.VMEM((BPW,), jnp.int32),
        pltpu.VMEM((BPW,), jnp.int32),
        pltpu.VMEM((BPW,), jnp.int32),
        pltpu.VMEM((BPW,), jnp.int32),
        pltpu.VMEM((BPW,), jnp.int32),
        pltpu.VMEM((BPW,), jnp.int32),
        pltpu.VMEM((BPW,), jnp.int32),
        pltpu.VMEM((CH, 128), jnp.float32),
        pltpu.VMEM((CH, 128), jnp.float32),
        pltpu.VMEM((CH, 128), jnp.float32),
        pltpu.VMEM((CH, 128), jnp.float32),
        pltpu.VMEM((CH, 128), jnp.float32),
        pltpu.SemaphoreType.DMA,
    ],
)
def _sc_gather(user_ids, item_ids, mlp_user, mlp_item, mf_user, mf_item,
               out, uidx, iidx, ur, ir, fur, fir, uo, io, fuo, fio,
               ubuf, ibuf, fubuf, fibuf, comp, sem):
    wid = lax.axis_index("s") * NC + lax.axis_index("c")
    base = wid * BPW
    pltpu.sync_copy(user_ids.at[pl.ds(base, BPW)], uidx)
    pltpu.sync_copy(item_ids.at[pl.ds(base, BPW)], iidx)
    for t in range(BPW // L):
        sl = pl.ds(t * L, L)
        u = uidx[sl]
        i = iidx[sl]
        r, o = _split_mlp(u)
        ur[sl], uo[sl] = r, o
        r, o = _split_mlp(i)
        ir[sl], io[sl] = r, o
        r, o = _split_mf(u)
        fur[sl], fuo[sl] = r, o
        r, o = _split_mf(i)
        fir[sl], fio[sl] = r, o

    def chunk_body(ch, _):
        cb = ch * CH
        csl = pl.ds(cb, CH)
        c0 = pltpu.async_copy(mlp_user.at[ur.at[csl]], ubuf, sem)
        c1 = pltpu.async_copy(mlp_item.at[ir.at[csl]], ibuf, sem)
        c2 = pltpu.async_copy(mf_user.at[fur.at[csl]], fubuf, sem)
        c3 = pltpu.async_copy(mf_item.at[fir.at[csl]], fibuf, sem)
        c0.wait()
        c1.wait()
        c2.wait()
        c3.wait()
        for blk in range(CH // L):
            lrow = blk * L + _iota16()
            bsl = pl.ds(cb + blk * L, L)
            offu = uo[bsl]
            offi = io[bsl]
            offfu = fuo[bsl]
            offfi = fio[bsl]
            for c in range(MLP_HALF):
                cv = jnp.full((L,), c, jnp.int32)
                plsc.store_scatter(comp, [lrow, cv],
                                   plsc.load_gather(ubuf, [lrow, offu + c]))
                plsc.store_scatter(comp, [lrow, cv + 32],
                                   plsc.load_gather(ibuf, [lrow, offi + c]))
            for c in range(MF_DIM):
                cv = jnp.full((L,), c, jnp.int32)
                plsc.store_scatter(comp, [lrow, cv + 64],
                                   plsc.load_gather(fubuf, [lrow, offfu + c]))
                plsc.store_scatter(comp, [lrow, cv + 80],
                                   plsc.load_gather(fibuf, [lrow, offfi + c]))
        pltpu.sync_copy(comp, out.at[pl.ds(base + cb, CH)])
        return _

    lax.fori_loop(0, NCH, chunk_body, 0)


BT = 2048


def _tc_body(p_ref, W1_ref, b1_ref, W2_ref, b2_ref,
             W3_ref, b3_ref, W4_ref, b4_ref, out_ref):
    x = p_ref[:, 0:64]
    h = jnp.maximum(jnp.dot(x, W1_ref[...], preferred_element_type=jnp.float32)
                    + b1_ref[...], 0.0)
    h = jnp.maximum(jnp.dot(h, W2_ref[...], preferred_element_type=jnp.float32)
                    + b2_ref[...], 0.0)
    h = jnp.maximum(jnp.dot(h, W3_ref[...], preferred_element_type=jnp.float32)
                    + b3_ref[...], 0.0)
    mf = jnp.sum(p_ref[:, 64:80] * p_ref[:, 80:96], axis=1, keepdims=True)
    z = (mf * W4_ref[0:1, :]
         + jnp.dot(h, W4_ref[1:9, :], preferred_element_type=jnp.float32)
         + b4_ref[...])
    out_ref[...] = 1.0 / (1.0 + jnp.exp(-z))


def _tc_mlp(packed, W1, b1r, W2, b2r, W3, b3r, W4p, b4r):
    grid = (B // BT,)
    full = lambda g: (0, 0)
    return pl.pallas_call(
        _tc_body,
        grid=grid,
        in_specs=[
            pl.BlockSpec((BT, 128), lambda g: (g, 0)),
            pl.BlockSpec((64, 32), full),
            pl.BlockSpec((1, 32), full),
            pl.BlockSpec((32, 16), full),
            pl.BlockSpec((1, 16), full),
            pl.BlockSpec((16, 8), full),
            pl.BlockSpec((1, 8), full),
            pl.BlockSpec((16, 1), full),
            pl.BlockSpec((1, 1), full),
        ],
        out_specs=pl.BlockSpec((BT, 1), lambda g: (g, 0)),
        out_shape=jax.ShapeDtypeStruct((B, 1), jnp.float32),
    )(packed, W1, b1r, W2, b2r, W3, b3r, W4p, b4r)


def kernel(user_ids, item_ids, mf_user, mf_item, mlp_user, mlp_item,
           W1, b1, W2, b2, W3, b3, W4, b4):
    mlp_user_p, mlp_item_p, mf_user_p, mf_item_p = _relayout(
        mlp_user.T, mlp_item.T, mf_user.T, mf_item.T)
    packed = _sc_gather(user_ids, item_ids,
                        mlp_user_p, mlp_item_p, mf_user_p, mf_item_p)
    W4p = jnp.pad(W4, ((0, 7), (0, 0)))
    return _tc_mlp(packed, W1, b1.reshape(1, 32), W2, b2.reshape(1, 16),
                   W3, b3.reshape(1, 8), W4p, b4.reshape(1, 1))


# speedup vs baseline: 5.7697x; 1.0343x over previous
import functools

import jax
import jax.numpy as jnp
from jax import lax
from jax.experimental import pallas as pl
from jax.experimental.pallas import tpu as pltpu
from jax.experimental.pallas import tpu_sc as plsc

B = 16384
N_ROWS = 1000000
MF_DIM = 16
MLP_HALF = 32
QMLP = 2048 * 122
QMF = 2048 * 61
RMLP = 2048 * 123
RMF = 2048 * 62
NC = 2
NS = 16
NW = NC * NS
BPW = B // NW
CH = 64
NCH = BPW // CH
L = 16


TCOL = 2048


def _tr_mlp_body(mu0, mu1, mu2, mu3, mi0, mi1, mi2, mi3, muo, mio):
    muo[...] = jnp.transpose(jnp.concatenate(
        [r[...] for r in (mu0, mu1, mu2, mu3)], axis=0))
    mio[...] = jnp.transpose(jnp.concatenate(
        [r[...] for r in (mi0, mi1, mi2, mi3)], axis=0))


def _tr_mf_body(fu0, fu1, fu2, fu3, fu4, fu5, fu6, fu7,
                fi0, fi1, fi2, fi3, fi4, fi5, fi6, fi7, fuo, fio):
    fuo[...] = jnp.transpose(jnp.concatenate(
        [r[...] for r in (fu0, fu1, fu2, fu3, fu4, fu5, fu6, fu7)], axis=0))
    fio[...] = jnp.transpose(jnp.concatenate(
        [r[...] for r in (fi0, fi1, fi2, fi3, fi4, fi5, fi6, fi7)], axis=0))


def _relayout(mlp_user_t, mlp_item_t, mf_user_t, mf_item_t):
    ngrid_mlp = RMLP // TCOL
    ngrid_mf = RMF // TCOL
    mlp_specs = [pl.BlockSpec((MLP_HALF, TCOL),
                              lambda g, j=j: (0, g + 122 * j))
                 for j in range(4)]
    mf_specs = [pl.BlockSpec((MF_DIM, TCOL),
                             lambda g, j=j: (0, g + 61 * j))
                for j in range(8)]
    out_row = lambda g: (g, 0)
    tr_params = pltpu.CompilerParams(fuse_transposed_lhs_in_matmul=True)
    mlp_user_p, mlp_item_p = pl.pallas_call(
        _tr_mlp_body,
        grid=(ngrid_mlp,),
        compiler_params=tr_params,
        in_specs=mlp_specs + mlp_specs,
        out_specs=[pl.BlockSpec((TCOL, 128), out_row)] * 2,
        out_shape=[jax.ShapeDtypeStruct((RMLP, 128), jnp.float32)] * 2,
    )(*([mlp_user_t] * 4 + [mlp_item_t] * 4))
    mf_user_p, mf_item_p = pl.pallas_call(
        _tr_mf_body,
        grid=(ngrid_mf,),
        compiler_params=tr_params,
        in_specs=mf_specs + mf_specs,
        out_specs=[pl.BlockSpec((TCOL, 128), out_row)] * 2,
        out_shape=[jax.ShapeDtypeStruct((RMF, 128), jnp.float32)] * 2,
    )(*([mf_user_t] * 8 + [mf_item_t] * 8))
    return mlp_user_p, mlp_item_p, mf_user_p, mf_item_p


_mesh = plsc.VectorSubcoreMesh(core_axis_name="c", subcore_axis_name="s")


def _iota16():
    return lax.iota(jnp.int32, L)


def _split_mlp(ids):
    j = ((ids >= QMLP).astype(jnp.int32) + (ids >= 2 * QMLP).astype(jnp.int32)
         + (ids >= 3 * QMLP).astype(jnp.int32))
    return ids - j * QMLP, j * MLP_HALF


def _split_mf(ids):
    j = (ids >= QMF).astype(jnp.int32)
    for k in range(2, 8):
        j = j + (ids >= k * QMF).astype(jnp.int32)
    return ids - j * QMF, j * MF_DIM


@functools.partial(
    pl.kernel,
    mesh=_mesh,
    compiler_params=pltpu.CompilerParams(needs_layout_passes=False),
    out_type=jax.ShapeDtypeStruct((B, 128), jnp.float32),
    scratch_types=[
        pltpu.VMEM((BPW,), jnp.int32),
        pltpu.VMEM((BPW,), jnp.int32),
        pltpu.VMEM((BPW,), jnp.int32),
        pltpu.VMEM((BPW,), jnp.int32),
        pltpu.VMEM((BPW,), jnp.int32),
        pltpu.VMEM((BPW,), jnp.int32),
        pltpu.VMEM((BPW,), jnp.int32),
        pltpu.VMEM((BPW,), jnp.int32),
        pltpu.VMEM((BPW,), jnp.int32),
        pltpu.VMEM((BPW,), jnp.int32),
        pltpu.VMEM((2 * CH, 128), jnp.float32),
        pltpu.VMEM((2 * CH, 128), jnp.float32),
        pltpu.VMEM((2 * CH, 128), jnp.float32),
        pltpu.VMEM((2 * CH, 128), jnp.float32),
        pltpu.VMEM((CH, 128), jnp.float32),
        pltpu.SemaphoreType.DMA,
        pltpu.SemaphoreType.DMA,
    ],
)
def _sc_gather(user_ids, item_ids, mlp_user, mlp_item, mf_user, mf_item,
               out, uidx, iidx, ur, ir, fur, fir, uo, io, fuo, fio,
               ubuf, ibuf, fubuf, fibuf, comp, sem0, sem1):
    wid = lax.axis_index("s") * NC + lax.axis_index("c")
    base = wid * BPW
    pltpu.sync_copy(user_ids.at[pl.ds(base, BPW)], uidx)
    pltpu.sync_copy(item_ids.at[pl.ds(base, BPW)], iidx)
    for t in range(BPW // L):
        sl = pl.ds(t * L, L)
        u = uidx[sl]
        i = iidx[sl]
        r, o = _split_mlp(u)
        ur[sl], uo[sl] = r, o
        r, o = _split_mlp(i)
        ir[sl], io[sl] = r, o
        r, o = _split_mf(u)
        fur[sl], fuo[sl] = r, o
        r, o = _split_mf(i)
        fir[sl], fio[sl] = r, o

    def _fire(ch, sem):
        csl = pl.ds(ch * CH, CH)
        dst = pl.ds((ch & 1) * CH, CH)
        pltpu.async_copy(mlp_user.at[ur.at[csl]], ubuf.at[dst], sem)
        pltpu.async_copy(mlp_item.at[ir.at[csl]], ibuf.at[dst], sem)
        pltpu.async_copy(mf_user.at[fur.at[csl]], fubuf.at[dst], sem)
        pltpu.async_copy(mf_item.at[fir.at[csl]], fibuf.at[dst], sem)

    def _drain(par, sem):
        for buf in (ubuf, ibuf, fubuf, fibuf):
            pltpu.make_async_copy(out.at[pl.ds(0, CH)],
                                  buf.at[pl.ds(par * CH, CH)], sem).wait()

    _fire(0, sem0)

    def chunk_body(ch, carry):
        cb = ch * CH

        @pl.when((ch & 1) == 0)
        def _():
            pl.when(ch + 1 < NCH)(lambda: _fire(ch + 1, sem1))
            _drain(0, sem0)

        @pl.when((ch & 1) == 1)
        def _():
            pl.when(ch + 1 < NCH)(lambda: _fire(ch + 1, sem0))
            _drain(1, sem1)

        pbase = (ch & 1) * CH
        for blk in range(CH // L):
            lrow = pbase + blk * L + _iota16()
            crow = blk * L + _iota16()
            bsl = pl.ds(cb + blk * L, L)
            offu = uo[bsl]
            offi = io[bsl]
            offfu = fuo[bsl]
            offfi = fio[bsl]
            for c in range(MLP_HALF):
                cv = jnp.full((L,), c, jnp.int32)
                plsc.store_scatter(comp, [crow, cv],
                                   plsc.load_gather(ubuf, [lrow, offu + c]))
                plsc.store_scatter(comp, [crow, cv + 32],
                                   plsc.load_gather(ibuf, [lrow, offi + c]))
            for c in range(MF_DIM):
                cv = jnp.full((L,), c, jnp.int32)
                plsc.store_scatter(comp, [crow, cv + 64],
                                   plsc.load_gather(fubuf, [lrow, offfu + c]))
                plsc.store_scatter(comp, [crow, cv + 80],
                                   plsc.load_gather(fibuf, [lrow, offfi + c]))
        pltpu.sync_copy(comp, out.at[pl.ds(base + cb, CH)])
        return carry

    lax.fori_loop(0, NCH, chunk_body, 0)


BT = 2048


def _tc_body(p_ref, W1_ref, b1_ref, W2_ref, b2_ref,
             W3_ref, b3_ref, W4_ref, b4_ref, out_ref):
    x = p_ref[:, 0:64]
    h = jnp.maximum(jnp.dot(x, W1_ref[...], preferred_element_type=jnp.float32)
                    + b1_ref[...], 0.0)
    h = jnp.maximum(jnp.dot(h, W2_ref[...], preferred_element_type=jnp.float32)
                    + b2_ref[...], 0.0)
    h = jnp.maximum(jnp.dot(h, W3_ref[...], preferred_element_type=jnp.float32)
                    + b3_ref[...], 0.0)
    mf = jnp.sum(p_ref[:, 64:80] * p_ref[:, 80:96], axis=1, keepdims=True)
    z = (mf * W4_ref[0:1, :]
         + jnp.dot(h, W4_ref[1:9, :], preferred_element_type=jnp.float32)
         + b4_ref[...])
    out_ref[...] = 1.0 / (1.0 + jnp.exp(-z))


def _tc_mlp(packed, W1, b1r, W2, b2r, W3, b3r, W4p, b4r):
    grid = (B // BT,)
    full = lambda g: (0, 0)
    return pl.pallas_call(
        _tc_body,
        grid=grid,
        in_specs=[
            pl.BlockSpec((BT, 128), lambda g: (g, 0)),
            pl.BlockSpec((64, 32), full),
            pl.BlockSpec((1, 32), full),
            pl.BlockSpec((32, 16), full),
            pl.BlockSpec((1, 16), full),
            pl.BlockSpec((16, 8), full),
            pl.BlockSpec((1, 8), full),
            pl.BlockSpec((16, 1), full),
            pl.BlockSpec((1, 1), full),
        ],
        out_specs=pl.BlockSpec((BT, 1), lambda g: (g, 0)),
        out_shape=jax.ShapeDtypeStruct((B, 1), jnp.float32),
    )(packed, W1, b1r, W2, b2r, W3, b3r, W4p, b4r)


def kernel(user_ids, item_ids, mf_user, mf_item, mlp_user, mlp_item,
           W1, b1, W2, b2, W3, b3, W4, b4):
    mlp_user_p, mlp_item_p, mf_user_p, mf_item_p = _relayout(
        mlp_user.T, mlp_item.T, mf_user.T, mf_item.T)
    packed = _sc_gather(user_ids, item_ids,
                        mlp_user_p, mlp_item_p, mf_user_p, mf_item_p)
    W4p = jnp.pad(W4, ((0, 7), (0, 0)))
    return _tc_mlp(packed, W1, b1.reshape(1, 32), W2, b2.reshape(1, 16),
                   W3, b3.reshape(1, 8), W4p, b4.reshape(1, 1))


# speedup vs baseline: 6.2705x; 1.0868x over previous
import functools

import jax
import jax.numpy as jnp
from jax import lax
from jax.experimental import pallas as pl
from jax.experimental.pallas import tpu as pltpu
from jax.experimental.pallas import tpu_sc as plsc

B = 16384
N_ROWS = 1000000
MF_DIM = 16
MLP_HALF = 32
QMLP = 2048 * 122
QMF = 2048 * 61
RMLP = 2048 * 123
RMF = 2048 * 62
NC = 2
NS = 16
NW = NC * NS
BPW = B // NW
CH = 64
NCH = BPW // CH
L = 16


TCOL = 2048


def _tr_mlp_body(mu0, mu1, mu2, mu3, mi0, mi1, mi2, mi3, muo, mio):
    muo[...] = jnp.transpose(jnp.concatenate(
        [r[...] for r in (mu0, mu1, mu2, mu3)], axis=0))
    mio[...] = jnp.transpose(jnp.concatenate(
        [r[...] for r in (mi0, mi1, mi2, mi3)], axis=0))


def _tr_mf_body(fu0, fu1, fu2, fu3, fu4, fu5, fu6, fu7,
                fi0, fi1, fi2, fi3, fi4, fi5, fi6, fi7, fuo, fio):
    fuo[...] = jnp.transpose(jnp.concatenate(
        [r[...] for r in (fu0, fu1, fu2, fu3, fu4, fu5, fu6, fu7)], axis=0))
    fio[...] = jnp.transpose(jnp.concatenate(
        [r[...] for r in (fi0, fi1, fi2, fi3, fi4, fi5, fi6, fi7)], axis=0))


_out_row = lambda g: (g, 0)


def _relayout_mlp(mlp_user_t, mlp_item_t):
    specs = [pl.BlockSpec((MLP_HALF, TCOL), lambda g, j=j: (0, g + 122 * j))
             for j in range(4)]
    return pl.pallas_call(
        _tr_mlp_body,
        grid=(RMLP // TCOL,),
        in_specs=specs + specs,
        out_specs=[pl.BlockSpec((TCOL, 128), _out_row)] * 2,
        out_shape=[jax.ShapeDtypeStruct((RMLP, 128), jnp.float32)] * 2,
    )(*([mlp_user_t] * 4 + [mlp_item_t] * 4))


def _relayout_mf(mf_user_t, mf_item_t):
    specs = [pl.BlockSpec((MF_DIM, TCOL), lambda g, j=j: (0, g + 61 * j))
             for j in range(8)]
    return pl.pallas_call(
        _tr_mf_body,
        grid=(RMF // TCOL,),
        in_specs=specs + specs,
        out_specs=[pl.BlockSpec((TCOL, 128), _out_row)] * 2,
        out_shape=[jax.ShapeDtypeStruct((RMF, 128), jnp.float32)] * 2,
    )(*([mf_user_t] * 8 + [mf_item_t] * 8))


_mesh = plsc.VectorSubcoreMesh(core_axis_name="c", subcore_axis_name="s")


def _iota16():
    return lax.iota(jnp.int32, L)


def _split_mlp(ids):
    j = ((ids >= QMLP).astype(jnp.int32) + (ids >= 2 * QMLP).astype(jnp.int32)
         + (ids >= 3 * QMLP).astype(jnp.int32))
    return ids - j * QMLP, j * MLP_HALF


def _split_mf(ids):
    j = (ids >= QMF).astype(jnp.int32)
    for k in range(2, 8):
        j = j + (ids >= k * QMF).astype(jnp.int32)
    return ids - j * QMF, j * MF_DIM


def _make_sc_gather(F, split_fn):
    @functools.partial(
        pl.kernel,
        mesh=_mesh,
        compiler_params=pltpu.CompilerParams(needs_layout_passes=False),
        out_type=jax.ShapeDtypeStruct((B, 2 * F), jnp.float32),
        scratch_types=[
            pltpu.VMEM((BPW,), jnp.int32),
            pltpu.VMEM((BPW,), jnp.int32),
            pltpu.VMEM((BPW,), jnp.int32),
            pltpu.VMEM((BPW,), jnp.int32),
            pltpu.VMEM((BPW,), jnp.int32),
            pltpu.VMEM((BPW,), jnp.int32),
            pltpu.VMEM((2 * CH, 128), jnp.float32),
            pltpu.VMEM((2 * CH, 128), jnp.float32),
            pltpu.VMEM((CH, 2 * F), jnp.float32),
            pltpu.SemaphoreType.DMA,
            pltpu.SemaphoreType.DMA,
        ],
    )
    def sc_gather(user_ids, item_ids, tbl_u, tbl_i,
                  out, uidx, iidx, ur, ir, uo, io, ubuf, ibuf, comp,
                  sem0, sem1):
        wid = lax.axis_index("s") * NC + lax.axis_index("c")
        base = wid * BPW
        pltpu.sync_copy(user_ids.at[pl.ds(base, BPW)], uidx)
        pltpu.sync_copy(item_ids.at[pl.ds(base, BPW)], iidx)
        for t in range(BPW // L):
            sl = pl.ds(t * L, L)
            r, o = split_fn(uidx[sl])
            ur[sl], uo[sl] = r, o
            r, o = split_fn(iidx[sl])
            ir[sl], io[sl] = r, o

        def _fire(ch, sem):
            csl = pl.ds(ch * CH, CH)
            dst = pl.ds((ch & 1) * CH, CH)
            pltpu.async_copy(tbl_u.at[ur.at[csl]], ubuf.at[dst], sem)
            pltpu.async_copy(tbl_i.at[ir.at[csl]], ibuf.at[dst], sem)

        def _drain(par, sem):
            for buf in (ubuf, ibuf):
                pltpu.make_async_copy(tbl_u.at[pl.ds(0, CH)],
                                      buf.at[pl.ds(par * CH, CH)], sem).wait()

        _fire(0, sem0)

        def chunk_body(ch, carry):
            cb = ch * CH

            @pl.when((ch & 1) == 0)
            def _():
                pl.when(ch + 1 < NCH)(lambda: _fire(ch + 1, sem1))
                _drain(0, sem0)

            @pl.when((ch & 1) == 1)
            def _():
                pl.when(ch + 1 < NCH)(lambda: _fire(ch + 1, sem0))
                _drain(1, sem1)

            pbase = (ch & 1) * CH
            for blk in range(CH // L):
                lrow = pbase + blk * L + _iota16()
                crow = blk * L + _iota16()
                bsl = pl.ds(cb + blk * L, L)
                offu = uo[bsl]
                offi = io[bsl]
                for c in range(F):
                    cv = jnp.full((L,), c, jnp.int32)
                    plsc.store_scatter(comp, [crow, cv],
                                       plsc.load_gather(ubuf, [lrow, offu + c]))
                    plsc.store_scatter(comp, [crow, cv + F],
                                       plsc.load_gather(ibuf, [lrow, offi + c]))
            pltpu.sync_copy(comp, out.at[pl.ds(base + cb, CH)])
            return carry

        lax.fori_loop(0, NCH, chunk_body, 0)

    return sc_gather


_sc_gather_mlp = _make_sc_gather(MLP_HALF, _split_mlp)
_sc_gather_mf = _make_sc_gather(MF_DIM, _split_mf)


BT = 2048


def _tc_body(p1_ref, p2_ref, W1_ref, b1_ref, W2_ref, b2_ref,
             W3_ref, b3_ref, W4_ref, b4_ref, out_ref):
    x = p1_ref[...]
    h = jnp.maximum(jnp.dot(x, W1_ref[...], preferred_element_type=jnp.float32)
                    + b1_ref[...], 0.0)
    h = jnp.maximum(jnp.dot(h, W2_ref[...], preferred_element_type=jnp.float32)
                    + b2_ref[...], 0.0)
    h = jnp.maximum(jnp.dot(h, W3_ref[...], preferred_element_type=jnp.float32)
                    + b3_ref[...], 0.0)
    mf = jnp.sum(p2_ref[:, 0:16] * p2_ref[:, 16:32], axis=1, keepdims=True)
    z = (mf * W4_ref[0:1, :]
         + jnp.dot(h, W4_ref[1:9, :], preferred_element_type=jnp.float32)
         + b4_ref[...])
    out_ref[...] = 1.0 / (1.0 + jnp.exp(-z))


def _tc_mlp(pmlp, pmf, W1, b1r, W2, b2r, W3, b3r, W4p, b4r):
    grid = (B // BT,)
    full = lambda g: (0, 0)
    tile = lambda g: (g, 0)
    return pl.pallas_call(
        _tc_body,
        grid=grid,
        in_specs=[
            pl.BlockSpec((BT, 64), tile),
            pl.BlockSpec((BT, 32), tile),
            pl.BlockSpec((64, 32), full),
            pl.BlockSpec((1, 32), full),
            pl.BlockSpec((32, 16), full),
            pl.BlockSpec((1, 16), full),
            pl.BlockSpec((16, 8), full),
            pl.BlockSpec((1, 8), full),
            pl.BlockSpec((16, 1), full),
            pl.BlockSpec((1, 1), full),
        ],
        out_specs=pl.BlockSpec((BT, 1), tile),
        out_shape=jax.ShapeDtypeStruct((B, 1), jnp.float32),
    )(pmlp, pmf, W1, b1r, W2, b2r, W3, b3r, W4p, b4r)


def kernel(user_ids, item_ids, mf_user, mf_item, mlp_user, mlp_item,
           W1, b1, W2, b2, W3, b3, W4, b4):
    mlp_user_p, mlp_item_p = _relayout_mlp(mlp_user.T, mlp_item.T)
    pmlp = _sc_gather_mlp(user_ids, item_ids, mlp_user_p, mlp_item_p)
    mf_user_p, mf_item_p = _relayout_mf(mf_user.T, mf_item.T)
    pmf = _sc_gather_mf(user_ids, item_ids, mf_user_p, mf_item_p)
    W4p = jnp.pad(W4, ((0, 7), (0, 0)))
    return _tc_mlp(pmlp, pmf, W1, b1.reshape(1, 32), W2, b2.reshape(1, 16),
                   W3, b3.reshape(1, 8), W4p, b4.reshape(1, 1))


# speedup vs baseline: 6.8908x; 1.0989x over previous
import functools

import jax
import jax.numpy as jnp
from jax import lax
from jax.experimental import pallas as pl
from jax.experimental.pallas import tpu as pltpu
from jax.experimental.pallas import tpu_sc as plsc

B = 16384
N_ROWS = 1000000
MF_DIM = 16
MLP_HALF = 32
QMLP = 4096 * 61
QMF = 4096 * 30
RMLP = 4096 * 62
RMF = 4096 * 35
NC = 2
NS = 16
NW = NC * NS
BPW = B // NW
CH = 64
NCH = BPW // CH
L = 16


TCOL = 4096


def _tr_mlp_body(mu0, mu1, mu2, mu3, mi0, mi1, mi2, mi3, muo, mio):
    muo[...] = jnp.transpose(jnp.concatenate(
        [r[...] for r in (mu0, mu1, mu2, mu3)], axis=0))
    mio[...] = jnp.transpose(jnp.concatenate(
        [r[...] for r in (mi0, mi1, mi2, mi3)], axis=0))


def _tr_mf_body(fu0, fu1, fu2, fu3, fu4, fu5, fu6, fu7,
                fi0, fi1, fi2, fi3, fi4, fi5, fi6, fi7, fuo, fio):
    fuo[...] = jnp.transpose(jnp.concatenate(
        [r[...] for r in (fu0, fu1, fu2, fu3, fu4, fu5, fu6, fu7)], axis=0))
    fio[...] = jnp.transpose(jnp.concatenate(
        [r[...] for r in (fi0, fi1, fi2, fi3, fi4, fi5, fi6, fi7)], axis=0))


_out_row = lambda g: (g, 0)


def _relayout_mlp(mlp_user_t, mlp_item_t):
    specs = [pl.BlockSpec((MLP_HALF, TCOL), lambda g, j=j: (0, g + 61 * j))
             for j in range(4)]
    return pl.pallas_call(
        _tr_mlp_body,
        grid=(RMLP // TCOL,),
        in_specs=specs + specs,
        out_specs=[pl.BlockSpec((TCOL, 128), _out_row)] * 2,
        out_shape=[jax.ShapeDtypeStruct((RMLP, 128), jnp.float32)] * 2,
    )(*([mlp_user_t] * 4 + [mlp_item_t] * 4))


def _relayout_mf(mf_user_t, mf_item_t):
    specs = [pl.BlockSpec((MF_DIM, TCOL), lambda g, j=j: (0, g + 30 * j))
             for j in range(8)]
    return pl.pallas_call(
        _tr_mf_body,
        grid=(RMF // TCOL,),
        in_specs=specs + specs,
        out_specs=[pl.BlockSpec((TCOL, 128), _out_row)] * 2,
        out_shape=[jax.ShapeDtypeStruct((RMF, 128), jnp.float32)] * 2,
    )(*([mf_user_t] * 8 + [mf_item_t] * 8))


_mesh = plsc.VectorSubcoreMesh(core_axis_name="c", subcore_axis_name="s")


def _iota16():
    return lax.iota(jnp.int32, L)


def _split_mlp(ids):
    j = ((ids >= QMLP).astype(jnp.int32) + (ids >= 2 * QMLP).astype(jnp.int32)
         + (ids >= 3 * QMLP).astype(jnp.int32))
    return ids - j * QMLP, j * MLP_HALF


def _split_mf(ids):
    j = (ids >= QMF).astype(jnp.int32)
    for k in range(2, 8):
        j = j + (ids >= k * QMF).astype(jnp.int32)
    return ids - j * QMF, j * MF_DIM


def _make_sc_gather(F, split_fn):
    @functools.partial(
        pl.kernel,
        mesh=_mesh,
        compiler_params=pltpu.CompilerParams(needs_layout_passes=False),
        out_type=jax.ShapeDtypeStruct((B, 2 * F), jnp.float32),
        scratch_types=[
            pltpu.VMEM((BPW,), jnp.int32),
            pltpu.VMEM((BPW,), jnp.int32),
            pltpu.VMEM((BPW,), jnp.int32),
            pltpu.VMEM((BPW,), jnp.int32),
            pltpu.VMEM((BPW,), jnp.int32),
            pltpu.VMEM((BPW,), jnp.int32),
            pltpu.VMEM((2 * CH, 128), jnp.float32),
            pltpu.VMEM((2 * CH, 128), jnp.float32),
            pltpu.VMEM((CH, 2 * F), jnp.float32),
            pltpu.SemaphoreType.DMA,
            pltpu.SemaphoreType.DMA,
        ],
    )
    def sc_gather(user_ids, item_ids, tbl_u, tbl_i,
                  out, uidx, iidx, ur, ir, uo, io, ubuf, ibuf, comp,
                  sem0, sem1):
        wid = lax.axis_index("s") * NC + lax.axis_index("c")
        base = wid * BPW
        pltpu.sync_copy(user_ids.at[pl.ds(base, BPW)], uidx)
        pltpu.sync_copy(item_ids.at[pl.ds(base, BPW)], iidx)
        for t in range(BPW // L):
            sl = pl.ds(t * L, L)
            r, o = split_fn(uidx[sl])
            ur[sl], uo[sl] = r, o
            r, o = split_fn(iidx[sl])
            ir[sl], io[sl] = r, o

        def _fire(ch, sem):
            csl = pl.ds(ch * CH, CH)
            dst = pl.ds((ch & 1) * CH, CH)
            pltpu.async_copy(tbl_u.at[ur.at[csl]], ubuf.at[dst], sem)
            pltpu.async_copy(tbl_i.at[ir.at[csl]], ibuf.at[dst], sem)

        def _drain(par, sem):
            for buf in (ubuf, ibuf):
                pltpu.make_async_copy(tbl_u.at[pl.ds(0, CH)],
                                      buf.at[pl.ds(par * CH, CH)], sem).wait()

        _fire(0, sem0)

        def chunk_body(ch, carry):
            cb = ch * CH

            @pl.when((ch & 1) == 0)
            def _():
                pl.when(ch + 1 < NCH)(lambda: _fire(ch + 1, sem1))
                _drain(0, sem0)

            @pl.when((ch & 1) == 1)
            def _():
                pl.when(ch + 1 < NCH)(lambda: _fire(ch + 1, sem0))
                _drain(1, sem1)

            pbase = (ch & 1) * CH
            for blk in range(CH // L):
                lrow = pbase + blk * L + _iota16()
                crow = blk * L + _iota16()
                bsl = pl.ds(cb + blk * L, L)
                offu = uo[bsl]
                offi = io[bsl]
                for c in range(F):
                    cv = jnp.full((L,), c, jnp.int32)
                    plsc.store_scatter(comp, [crow, cv],
                                       plsc.load_gather(ubuf, [lrow, offu + c]))
                    plsc.store_scatter(comp, [crow, cv + F],
                                       plsc.load_gather(ibuf, [lrow, offi + c]))
            pltpu.sync_copy(comp, out.at[pl.ds(base + cb, CH)])
            return carry

        lax.fori_loop(0, NCH, chunk_body, 0)

    return sc_gather


_sc_gather_mlp = _make_sc_gather(MLP_HALF, _split_mlp)
_sc_gather_mf = _make_sc_gather(MF_DIM, _split_mf)


BT = 2048


def _tc_body(p1_ref, p2_ref, W1_ref, b1_ref, W2_ref, b2_ref,
             W3_ref, b3_ref, W4_ref, b4_ref, out_ref):
    x = p1_ref[...]
    h = jnp.maximum(jnp.dot(x, W1_ref[...], preferred_element_type=jnp.float32)
                    + b1_ref[...], 0.0)
    h = jnp.maximum(jnp.dot(h, W2_ref[...], preferred_element_type=jnp.float32)
                    + b2_ref[...], 0.0)
    h = jnp.maximum(jnp.dot(h, W3_ref[...], preferred_element_type=jnp.float32)
                    + b3_ref[...], 0.0)
    mf = jnp.sum(p2_ref[:, 0:16] * p2_ref[:, 16:32], axis=1, keepdims=True)
    z = (mf * W4_ref[0:1, :]
         + jnp.dot(h, W4_ref[1:9, :], preferred_element_type=jnp.float32)
         + b4_ref[...])
    out_ref[...] = 1.0 / (1.0 + jnp.exp(-z))


def _tc_mlp(pmlp, pmf, W1, b1r, W2, b2r, W3, b3r, W4p, b4r):
    grid = (B // BT,)
    full = lambda g: (0, 0)
    tile = lambda g: (g, 0)
    return pl.pallas_call(
        _tc_body,
        grid=grid,
        in_specs=[
            pl.BlockSpec((BT, 64), tile),
            pl.BlockSpec((BT, 32), tile),
            pl.BlockSpec((64, 32), full),
            pl.BlockSpec((1, 32), full),
            pl.BlockSpec((32, 16), full),
            pl.BlockSpec((1, 16), full),
            pl.BlockSpec((16, 8), full),
            pl.BlockSpec((1, 8), full),
            pl.BlockSpec((16, 1), full),
            pl.BlockSpec((1, 1), full),
        ],
        out_specs=pl.BlockSpec((BT, 1), tile),
        out_shape=jax.ShapeDtypeStruct((B, 1), jnp.float32),
    )(pmlp, pmf, W1, b1r, W2, b2r, W3, b3r, W4p, b4r)


def kernel(user_ids, item_ids, mf_user, mf_item, mlp_user, mlp_item,
           W1, b1, W2, b2, W3, b3, W4, b4):
    mlp_user_p, mlp_item_p = _relayout_mlp(mlp_user.T, mlp_item.T)
    pmlp = _sc_gather_mlp(user_ids, item_ids, mlp_user_p, mlp_item_p)
    mf_user_p, mf_item_p = _relayout_mf(mf_user.T, mf_item.T)
    pmf = _sc_gather_mf(user_ids, item_ids, mf_user_p, mf_item_p)
    W4p = jnp.pad(W4, ((0, 7), (0, 0)))
    return _tc_mlp(pmlp, pmf, W1, b1.reshape(1, 32), W2, b2.reshape(1, 16),
                   W3, b3.reshape(1, 8), W4p, b4.reshape(1, 1))
